# baseline (speedup 1.0000x reference)
import functools

import jax
import jax.numpy as jnp
from jax import lax
from jax.experimental import pallas as pl
from jax.experimental.pallas import tpu as pltpu
from jax.experimental.pallas import tpu_sc as plsc

H = 1024
I = 4096
E = 64
K = 8
F = 512
N = 2048
NK = N * K
TM = 128
NTILES = NK // TM
NSTEPS = NTILES + E - 1

NC = 2
NS = 16
NW = NC * NS

NEGINF = -1e30


def _router_body(x_ref, wg_ref, logits_ref, w_ref, pos_ref, counts_ref):
    x = x_ref[...]
    wg = wg_ref[...]
    logits = jax.lax.dot_general(
        x, wg, (((1,), (1,)), ((), ())), preferred_element_type=jnp.float32
    )
    iota_e = jax.lax.broadcasted_iota(jnp.int32, (N, E), 1)
    cur = logits
    vals, sels = [], []
    mask = jnp.zeros((N, E), jnp.float32)
    for _ in range(K):
        m = jnp.max(cur, axis=1, keepdims=True)
        idx = jnp.min(jnp.where(cur == m, iota_e, E), axis=1, keepdims=True)
        vals.append(m)
        sels.append(idx)
        mask = mask + jnp.where(iota_e == idx, 1.0, 0.0)
        cur = jnp.where(iota_e == idx, NEGINF, cur)
    exps = [jnp.exp(v - vals[0]) for v in vals]
    denom = functools.reduce(jnp.add, exps)
    c = mask
    sh = 1
    while sh < N:
        c = c + jnp.concatenate(
            [jnp.zeros((sh, E), jnp.float32), c[: N - sh]], axis=0)
        sh *= 2
    rank = c - mask
    counts = c[N - 1:N, :]
    cc = counts
    sh = 1
    while sh < E:
        cc = cc + jnp.concatenate(
            [jnp.zeros((1, sh), jnp.float32), cc[:, : E - sh]], axis=1)
        sh *= 2
    offs = cc - counts
    posfull = offs + rank
    pos_cols = [
        jnp.sum(jnp.where(iota_e == sels[k], posfull, 0.0), axis=1,
                keepdims=True)
        for k in range(K)
    ]
    logits_ref[...] = logits
    w_ref[...] = jnp.concatenate([e / denom for e in exps], axis=1)
    pos_ref[...] = jnp.concatenate(pos_cols, axis=1).astype(jnp.int32)
    counts_ref[...] = counts.astype(jnp.int32)


def _shared_body(xb_ref, w1_ref, w2_ref, out_ref):
    i = pl.program_id(0)

    @pl.when(i == 0)
    def _():
        out_ref[...] = jnp.zeros_like(out_ref)

    xb = xb_ref[...]
    w1 = w1_ref[...].astype(jnp.bfloat16)
    h = jax.lax.dot_general(
        xb, w1, (((1,), (1,)), ((), ())), preferred_element_type=jnp.float32
    )
    h = h * jax.nn.sigmoid(h)
    w2 = w2_ref[...].astype(jnp.bfloat16)
    out_ref[...] += jax.lax.dot_general(
        h.astype(jnp.bfloat16), w2, (((1,), (1,)), ((), ())),
        preferred_element_type=jnp.float32,
    )


def _gmm_body(tile_ref, gid_ref, lo_ref, hi_ref, last_ref, xs_ref, w1_ref,
              w2_ref, out_ref, acc_ref):
    s = pl.program_id(0)
    lo = lo_ref[s]
    hi = hi_ref[s]
    tile = tile_ref[s]

    @pl.when(lo < hi)
    def _():
        xs32 = xs_ref[...]
        xlo = jax.lax.bitcast_convert_type(
            xs32 << 16, jnp.float32).astype(jnp.bfloat16)
        xhi = jax.lax.bitcast_convert_type(
            xs32 & jnp.int32(-65536), jnp.float32).astype(jnp.bfloat16)
        w1 = w1_ref[0].astype(jnp.bfloat16)
        h = jax.lax.dot_general(
            xlo, w1[:, : H // 2], (((1,), (1,)), ((), ())),
            preferred_element_type=jnp.float32)
        h += jax.lax.dot_general(
            xhi, w1[:, H // 2:], (((1,), (1,)), ((), ())),
            preferred_element_type=jnp.float32)
        h = h * jax.nn.sigmoid(h)
        o = jax.lax.dot_general(
            h.astype(jnp.bfloat16), w2_ref[0].astype(jnp.bfloat16),
            (((1,), (1,)), ((), ())), preferred_element_type=jnp.float32,
        )
        olo = jax.lax.bitcast_convert_type(
            o[:, : H // 2].astype(jnp.bfloat16).astype(jnp.float32),
            jnp.int32)
        ohi = jax.lax.bitcast_convert_type(
            o[:, H // 2:].astype(jnp.bfloat16).astype(jnp.float32),
            jnp.int32)
        o32 = ((olo >> 16) & 65535) | (ohi & jnp.int32(-65536))
        r = jax.lax.broadcasted_iota(jnp.int32, (TM, H // 2), 0) + tile * TM
        sel = (r >= lo) & (r < hi)
        acc_ref[...] = jnp.where(sel, o32, acc_ref[...])

    @pl.when(last_ref[s] == 1)
    def _():
        out_ref[...] = acc_ref[...]


def _combine_body(yg_ref, w_ref, shared_ref, out_ref):
    k = pl.program_id(1)

    @pl.when(k == 0)
    def _():
        out_ref[...] = shared_ref[...]

    iota_k = jax.lax.broadcasted_iota(jnp.int32, (w_ref.shape[0], K), 1)
    wk = jnp.sum(jnp.where(iota_k == k, w_ref[...], 0.0), axis=1,
                 keepdims=True)
    yg32 = yg_ref[...]
    flo = jax.lax.bitcast_convert_type(yg32 << 16, jnp.float32)
    fhi = jax.lax.bitcast_convert_type(yg32 & jnp.int32(-65536), jnp.float32)
    out_ref[...] += wk * jnp.concatenate([flo, fhi], axis=1)


def _sc_dispatch_rows(src, tok, pos, chunk=64):
    j = tok.shape[0]
    d = src.shape[1]
    per_w = j // NW
    mesh = plsc.VectorSubcoreMesh(core_axis_name="c", subcore_axis_name="s")

    @functools.partial(
        pl.kernel, mesh=mesh,
        out_type=jax.ShapeDtypeStruct((j, d), src.dtype),
        scratch_types=[
            pltpu.VMEM((chunk,), jnp.int32),
            pltpu.VMEM((chunk,), jnp.int32),
            pltpu.VMEM((chunk, d), src.dtype),
            pltpu.SemaphoreType.DMA,
        ],
    )
    def kern(src_hbm, tok_hbm, pos_hbm, out_hbm, tok_v, pos_v, rows_v, sem):
        wid = lax.axis_index("s") * NC + lax.axis_index("c")
        base = wid * per_w
        for c in range(per_w // chunk):
            off = base + c * chunk
            pltpu.sync_copy(tok_hbm.at[pl.ds(off, chunk)], tok_v)
            pltpu.sync_copy(pos_hbm.at[pl.ds(off, chunk)], pos_v)
            pltpu.async_copy(src_hbm.at[tok_v], rows_v, sem).wait()
            pltpu.async_copy(rows_v, out_hbm.at[pos_v], sem).wait()

    return kern(src, tok, pos)


def _sc_gather_rows(src, idx, chunk=64):
    j = idx.shape[0]
    d = src.shape[1]
    per_w = j // NW
    mesh = plsc.VectorSubcoreMesh(core_axis_name="c", subcore_axis_name="s")

    @functools.partial(
        pl.kernel, mesh=mesh,
        out_type=jax.ShapeDtypeStruct((j, d), src.dtype),
        scratch_types=[
            pltpu.VMEM((chunk,), jnp.int32),
            pltpu.VMEM((chunk, d), src.dtype),
            pltpu.SemaphoreType.DMA,
        ],
    )
    def kern(src_hbm, idx_hbm, out_hbm, idx_v, rows_v, sem):
        wid = lax.axis_index("s") * NC + lax.axis_index("c")
        base = wid * per_w
        for c in range(per_w // chunk):
            off = base + c * chunk
            pltpu.sync_copy(idx_hbm.at[pl.ds(off, chunk)], idx_v)
            pltpu.async_copy(src_hbm.at[idx_v], rows_v, sem).wait()
            pltpu.sync_copy(rows_v, out_hbm.at[pl.ds(off, chunk)])

    return kern(src, idx)


def _metadata(counts):
    ends = jnp.cumsum(counts)
    starts = ends - counts
    first_tile = starts // TM
    last_tile = jnp.where(counts > 0, (ends - 1) // TM, 0)
    span = jnp.where(counts > 0, last_tile - first_tile + 1, 0)
    cum = jnp.cumsum(span)
    total = cum[-1]
    sidx = jnp.arange(NSTEPS, dtype=jnp.int32)
    gid = jnp.searchsorted(cum, sidx, side="right").astype(jnp.int32)
    valid = sidx < total
    gid_last = jnp.searchsorted(cum, total - 1, side="right").astype(jnp.int32)
    gidc = jnp.where(valid, jnp.minimum(gid, E - 1), gid_last)
    local = sidx - (cum[gidc] - span[gidc])
    tile = jnp.where(valid, first_tile[gidc] + local, NTILES - 1).astype(jnp.int32)
    lo = jnp.where(valid, jnp.maximum(starts[gidc], tile * TM), 0).astype(jnp.int32)
    hi = jnp.where(valid, jnp.minimum(ends[gidc], (tile + 1) * TM), 0).astype(jnp.int32)
    tile_next = jnp.concatenate([tile[1:], tile[:1]])
    last = jnp.where(
        valid, ((sidx + 1 == total) | (tile_next != tile)).astype(jnp.int32), 0
    ).astype(jnp.int32)
    return tile, gidc, lo, hi, last


def kernel(hidden_states, Wg, W1s, W2s, W1, W2):
    b, s, h = hidden_states.shape
    x = hidden_states.reshape(-1, h)
    xb = x.astype(jnp.bfloat16)

    logits, w, pos, counts = pl.pallas_call(
        _router_body,
        out_shape=(
            jax.ShapeDtypeStruct((N, E), jnp.float32),
            jax.ShapeDtypeStruct((N, K), jnp.float32),
            jax.ShapeDtypeStruct((N, K), jnp.int32),
            jax.ShapeDtypeStruct((1, E), jnp.int32),
        ),
    )(x, Wg)

    IC = 512
    shared = pl.pallas_call(
        _shared_body,
        grid=(I // IC,),
        in_specs=[
            pl.BlockSpec((N, H), lambda i: (0, 0)),
            pl.BlockSpec((IC, H), lambda i: (i, 0)),
            pl.BlockSpec((H, IC), lambda i: (0, i)),
        ],
        out_specs=pl.BlockSpec((N, H), lambda i: (0, 0)),
        out_shape=jax.ShapeDtypeStruct((N, H), jnp.float32),
    )(xb, W1s, W2s)

    tok = jnp.arange(NK, dtype=jnp.int32) // K
    pos_flat = pos.reshape(NK)
    lo16 = jax.lax.bitcast_convert_type(
        xb[:, : H // 2], jnp.uint16).astype(jnp.uint32)
    hi16 = jax.lax.bitcast_convert_type(
        xb[:, H // 2:], jnp.uint16).astype(jnp.uint32)
    xb32 = jax.lax.bitcast_convert_type(lo16 | (hi16 << 16), jnp.int32)
    xs = _sc_dispatch_rows(xb32, tok, pos_flat)

    tile, gid, lo, hi, lastf = _metadata(counts[0])
    ys = pl.pallas_call(
        _gmm_body,
        grid_spec=pltpu.PrefetchScalarGridSpec(
            num_scalar_prefetch=5,
            grid=(NSTEPS,),
            in_specs=[
                pl.BlockSpec((TM, H // 2),
                             lambda st, t, g, l, hh, lf: (t[st], 0)),
                pl.BlockSpec((1, F, H),
                             lambda st, t, g, l, hh, lf: (g[st], 0, 0)),
                pl.BlockSpec((1, H, F),
                             lambda st, t, g, l, hh, lf: (g[st], 0, 0)),
            ],
            out_specs=pl.BlockSpec((TM, H // 2),
                                   lambda st, t, g, l, hh, lf: (t[st], 0)),
            scratch_shapes=[pltpu.VMEM((TM, H // 2), jnp.int32)],
        ),
        out_shape=jax.ShapeDtypeStruct((NK, H // 2), jnp.int32),
    )(tile, gid, lo, hi, lastf, xs, W1, W2)

    pos_kmaj = pos.T.reshape(NK)
    yg = _sc_gather_rows(ys, pos_kmaj)

    TN = 256
    out = pl.pallas_call(
        _combine_body,
        grid=(N // TN, K),
        in_specs=[
            pl.BlockSpec((TN, H // 2), lambda t, k: (k * (N // TN) + t, 0)),
            pl.BlockSpec((TN, K), lambda t, k: (t, 0)),
            pl.BlockSpec((TN, H), lambda t, k: (t, 0)),
        ],
        out_specs=pl.BlockSpec((TN, H), lambda t, k: (t, 0)),
        out_shape=jax.ShapeDtypeStruct((N, H), jnp.float32),
    )(yg, w, shared)

    return (out.reshape(b, s, h), logits.reshape(b, s, E))


# speedup vs baseline: 1.1520x; 1.1520x over previous
import functools

import jax
import jax.numpy as jnp
from jax import lax
from jax.experimental import pallas as pl
from jax.experimental.pallas import tpu as pltpu
from jax.experimental.pallas import tpu_sc as plsc

H = 1024
I = 4096
E = 64
K = 8
F = 512
N = 2048
NK = N * K
TM = 256
NTILES = NK // TM
NSTEPS = NTILES + E - 1

NC = 2
NS = 16
NW = NC * NS

NEGINF = -1e30


def _router_body(x_ref, wg_ref, logits_ref, w_ref, pos_ref, counts_ref):
    x = x_ref[...]
    wg = wg_ref[...]
    logits = jax.lax.dot_general(
        x, wg, (((1,), (1,)), ((), ())), preferred_element_type=jnp.float32
    )
    iota_e = jax.lax.broadcasted_iota(jnp.int32, (N, E), 1)
    cur = logits
    vals, sels = [], []
    mask = jnp.zeros((N, E), jnp.float32)
    for _ in range(K):
        m = jnp.max(cur, axis=1, keepdims=True)
        idx = jnp.min(jnp.where(cur == m, iota_e, E), axis=1, keepdims=True)
        vals.append(m)
        sels.append(idx)
        mask = mask + jnp.where(iota_e == idx, 1.0, 0.0)
        cur = jnp.where(iota_e == idx, NEGINF, cur)
    exps = [jnp.exp(v - vals[0]) for v in vals]
    denom = functools.reduce(jnp.add, exps)
    c = mask
    sh = 1
    while sh < N:
        c = c + jnp.concatenate(
            [jnp.zeros((sh, E), jnp.float32), c[: N - sh]], axis=0)
        sh *= 2
    rank = c - mask
    counts = c[N - 1:N, :]
    cc = counts
    sh = 1
    while sh < E:
        cc = cc + jnp.concatenate(
            [jnp.zeros((1, sh), jnp.float32), cc[:, : E - sh]], axis=1)
        sh *= 2
    offs = cc - counts
    posfull = offs + rank
    pos_cols = [
        jnp.sum(jnp.where(iota_e == sels[k], posfull, 0.0), axis=1,
                keepdims=True)
        for k in range(K)
    ]
    logits_ref[...] = logits
    w_ref[...] = jnp.concatenate([e / denom for e in exps], axis=1)
    pos_ref[...] = jnp.concatenate(pos_cols, axis=1).astype(jnp.int32)
    counts_ref[...] = counts.astype(jnp.int32)


def _shared_body(xb_ref, w1_ref, w2_ref, out_ref):
    i = pl.program_id(0)

    @pl.when(i == 0)
    def _():
        out_ref[...] = jnp.zeros_like(out_ref)

    xb = xb_ref[...]
    w1 = w1_ref[...].astype(jnp.bfloat16)
    h = jax.lax.dot_general(
        xb, w1, (((1,), (1,)), ((), ())), preferred_element_type=jnp.float32
    )
    h = h * jax.nn.sigmoid(h)
    w2 = w2_ref[...].astype(jnp.bfloat16)
    out_ref[...] += jax.lax.dot_general(
        h.astype(jnp.bfloat16), w2, (((1,), (1,)), ((), ())),
        preferred_element_type=jnp.float32,
    )


def _gmm_body(tile_ref, gid_ref, lo_ref, hi_ref, last_ref, xs_ref, w1_ref,
              w2_ref, out_ref, acc_ref):
    s = pl.program_id(0)
    lo = lo_ref[s]
    hi = hi_ref[s]
    tile = tile_ref[s]

    @pl.when(lo < hi)
    def _():
        xs32 = xs_ref[...]
        xlo = jax.lax.bitcast_convert_type(
            xs32 << 16, jnp.float32).astype(jnp.bfloat16)
        xhi = jax.lax.bitcast_convert_type(
            xs32 & jnp.int32(-65536), jnp.float32).astype(jnp.bfloat16)
        w1 = w1_ref[0].astype(jnp.bfloat16)
        h = jax.lax.dot_general(
            xlo, w1[:, : H // 2], (((1,), (1,)), ((), ())),
            preferred_element_type=jnp.float32)
        h += jax.lax.dot_general(
            xhi, w1[:, H // 2:], (((1,), (1,)), ((), ())),
            preferred_element_type=jnp.float32)
        h = h * jax.nn.sigmoid(h)
        o = jax.lax.dot_general(
            h.astype(jnp.bfloat16), w2_ref[0].astype(jnp.bfloat16),
            (((1,), (1,)), ((), ())), preferred_element_type=jnp.float32,
        )
        olo = jax.lax.bitcast_convert_type(
            o[:, : H // 2].astype(jnp.bfloat16).astype(jnp.float32),
            jnp.int32)
        ohi = jax.lax.bitcast_convert_type(
            o[:, H // 2:].astype(jnp.bfloat16).astype(jnp.float32),
            jnp.int32)
        o32 = ((olo >> 16) & 65535) | (ohi & jnp.int32(-65536))
        r = jax.lax.broadcasted_iota(jnp.int32, (TM, H // 2), 0) + tile * TM
        sel = (r >= lo) & (r < hi)
        acc_ref[...] = jnp.where(sel, o32, acc_ref[...])

    @pl.when(last_ref[s] == 1)
    def _():
        out_ref[...] = acc_ref[...]


def _combine_body(yg_ref, w_ref, shared_ref, out_ref):
    k = pl.program_id(1)

    @pl.when(k == 0)
    def _():
        out_ref[...] = shared_ref[...]

    iota_k = jax.lax.broadcasted_iota(jnp.int32, (w_ref.shape[0], K), 1)
    wk = jnp.sum(jnp.where(iota_k == k, w_ref[...], 0.0), axis=1,
                 keepdims=True)
    yg32 = yg_ref[...]
    flo = jax.lax.bitcast_convert_type(yg32 << 16, jnp.float32)
    fhi = jax.lax.bitcast_convert_type(yg32 & jnp.int32(-65536), jnp.float32)
    out_ref[...] += wk * jnp.concatenate([flo, fhi], axis=1)


def _sc_dispatch_rows(src, tok, pos, chunk=64):
    j = tok.shape[0]
    d = src.shape[1]
    per_w = j // NW
    mesh = plsc.VectorSubcoreMesh(core_axis_name="c", subcore_axis_name="s")

    @functools.partial(
        pl.kernel, mesh=mesh,
        out_type=jax.ShapeDtypeStruct((j, d), src.dtype),
        scratch_types=[
            pltpu.VMEM((chunk,), jnp.int32),
            pltpu.VMEM((chunk,), jnp.int32),
            pltpu.VMEM((chunk, d), src.dtype),
            pltpu.SemaphoreType.DMA,
        ],
    )
    def kern(src_hbm, tok_hbm, pos_hbm, out_hbm, tok_v, pos_v, rows_v, sem):
        wid = lax.axis_index("s") * NC + lax.axis_index("c")
        base = wid * per_w
        for c in range(per_w // chunk):
            off = base + c * chunk
            pltpu.sync_copy(tok_hbm.at[pl.ds(off, chunk)], tok_v)
            pltpu.sync_copy(pos_hbm.at[pl.ds(off, chunk)], pos_v)
            pltpu.async_copy(src_hbm.at[tok_v], rows_v, sem).wait()
            pltpu.async_copy(rows_v, out_hbm.at[pos_v], sem).wait()

    return kern(src, tok, pos)


def _sc_gather_rows(src, idx, chunk=64):
    j = idx.shape[0]
    d = src.shape[1]
    per_w = j // NW
    mesh = plsc.VectorSubcoreMesh(core_axis_name="c", subcore_axis_name="s")

    @functools.partial(
        pl.kernel, mesh=mesh,
        out_type=jax.ShapeDtypeStruct((j, d), src.dtype),
        scratch_types=[
            pltpu.VMEM((chunk,), jnp.int32),
            pltpu.VMEM((chunk, d), src.dtype),
            pltpu.SemaphoreType.DMA,
        ],
    )
    def kern(src_hbm, idx_hbm, out_hbm, idx_v, rows_v, sem):
        wid = lax.axis_index("s") * NC + lax.axis_index("c")
        base = wid * per_w
        for c in range(per_w // chunk):
            off = base + c * chunk
            pltpu.sync_copy(idx_hbm.at[pl.ds(off, chunk)], idx_v)
            pltpu.async_copy(src_hbm.at[idx_v], rows_v, sem).wait()
            pltpu.sync_copy(rows_v, out_hbm.at[pl.ds(off, chunk)])

    return kern(src, idx)


def _metadata(counts):
    ends = jnp.cumsum(counts)
    starts = ends - counts
    first_tile = starts // TM
    last_tile = jnp.where(counts > 0, (ends - 1) // TM, 0)
    span = jnp.where(counts > 0, last_tile - first_tile + 1, 0)
    cum = jnp.cumsum(span)
    total = cum[-1]
    sidx = jnp.arange(NSTEPS, dtype=jnp.int32)
    gid = jnp.searchsorted(cum, sidx, side="right").astype(jnp.int32)
    valid = sidx < total
    gid_last = jnp.searchsorted(cum, total - 1, side="right").astype(jnp.int32)
    gidc = jnp.where(valid, jnp.minimum(gid, E - 1), gid_last)
    local = sidx - (cum[gidc] - span[gidc])
    tile = jnp.where(valid, first_tile[gidc] + local, NTILES - 1).astype(jnp.int32)
    lo = jnp.where(valid, jnp.maximum(starts[gidc], tile * TM), 0).astype(jnp.int32)
    hi = jnp.where(valid, jnp.minimum(ends[gidc], (tile + 1) * TM), 0).astype(jnp.int32)
    tile_next = jnp.concatenate([tile[1:], tile[:1]])
    last = jnp.where(
        valid, ((sidx + 1 == total) | (tile_next != tile)).astype(jnp.int32), 0
    ).astype(jnp.int32)
    return tile, gidc, lo, hi, last


def kernel(hidden_states, Wg, W1s, W2s, W1, W2):
    b, s, h = hidden_states.shape
    x = hidden_states.reshape(-1, h)
    xb = x.astype(jnp.bfloat16)

    logits, w, pos, counts = pl.pallas_call(
        _router_body,
        out_shape=(
            jax.ShapeDtypeStruct((N, E), jnp.float32),
            jax.ShapeDtypeStruct((N, K), jnp.float32),
            jax.ShapeDtypeStruct((N, K), jnp.int32),
            jax.ShapeDtypeStruct((1, E), jnp.int32),
        ),
    )(x, Wg)

    IC = 512
    shared = pl.pallas_call(
        _shared_body,
        grid=(I // IC,),
        in_specs=[
            pl.BlockSpec((N, H), lambda i: (0, 0)),
            pl.BlockSpec((IC, H), lambda i: (i, 0)),
            pl.BlockSpec((H, IC), lambda i: (0, i)),
        ],
        out_specs=pl.BlockSpec((N, H), lambda i: (0, 0)),
        out_shape=jax.ShapeDtypeStruct((N, H), jnp.float32),
    )(xb, W1s, W2s)

    tok = jnp.arange(NK, dtype=jnp.int32) // K
    pos_flat = pos.reshape(NK)
    lo16 = jax.lax.bitcast_convert_type(
        xb[:, : H // 2], jnp.uint16).astype(jnp.uint32)
    hi16 = jax.lax.bitcast_convert_type(
        xb[:, H // 2:], jnp.uint16).astype(jnp.uint32)
    xb32 = jax.lax.bitcast_convert_type(lo16 | (hi16 << 16), jnp.int32)
    xs = _sc_dispatch_rows(xb32, tok, pos_flat)

    tile, gid, lo, hi, lastf = _metadata(counts[0])
    ys = pl.pallas_call(
        _gmm_body,
        grid_spec=pltpu.PrefetchScalarGridSpec(
            num_scalar_prefetch=5,
            grid=(NSTEPS,),
            in_specs=[
                pl.BlockSpec((TM, H // 2),
                             lambda st, t, g, l, hh, lf: (t[st], 0)),
                pl.BlockSpec((1, F, H),
                             lambda st, t, g, l, hh, lf: (g[st], 0, 0)),
                pl.BlockSpec((1, H, F),
                             lambda st, t, g, l, hh, lf: (g[st], 0, 0)),
            ],
            out_specs=pl.BlockSpec((TM, H // 2),
                                   lambda st, t, g, l, hh, lf: (t[st], 0)),
            scratch_shapes=[pltpu.VMEM((TM, H // 2), jnp.int32)],
        ),
        out_shape=jax.ShapeDtypeStruct((NK, H // 2), jnp.int32),
    )(tile, gid, lo, hi, lastf, xs, W1, W2)

    pos_kmaj = pos.T.reshape(NK)
    yg = _sc_gather_rows(ys, pos_kmaj)

    TN = 256
    out = pl.pallas_call(
        _combine_body,
        grid=(N // TN, K),
        in_specs=[
            pl.BlockSpec((TN, H // 2), lambda t, k: (k * (N // TN) + t, 0)),
            pl.BlockSpec((TN, K), lambda t, k: (t, 0)),
            pl.BlockSpec((TN, H), lambda t, k: (t, 0)),
        ],
        out_specs=pl.BlockSpec((TN, H), lambda t, k: (t, 0)),
        out_shape=jax.ShapeDtypeStruct((N, H), jnp.float32),
    )(yg, w, shared)

    return (out.reshape(b, s, h), logits.reshape(b, s, E))


# speedup vs baseline: 1.2286x; 1.0664x over previous
import functools

import jax
import jax.numpy as jnp
from jax import lax
from jax.experimental import pallas as pl
from jax.experimental.pallas import tpu as pltpu
from jax.experimental.pallas import tpu_sc as plsc

H = 1024
I = 4096
E = 64
K = 8
F = 512
N = 2048
NK = N * K
TM = 512
NTILES = NK // TM
NSTEPS = NTILES + E - 1

NC = 2
NS = 16
NW = NC * NS

NEGINF = -1e30


def _router_body(x_ref, wg_ref, logits_ref, w_ref, pos_ref, counts_ref):
    x = x_ref[...]
    wg = wg_ref[...]
    logits = jax.lax.dot_general(
        x, wg, (((1,), (1,)), ((), ())), preferred_element_type=jnp.float32
    )
    iota_e = jax.lax.broadcasted_iota(jnp.int32, (N, E), 1)
    cur = logits
    vals, sels = [], []
    mask = jnp.zeros((N, E), jnp.float32)
    for _ in range(K):
        m = jnp.max(cur, axis=1, keepdims=True)
        idx = jnp.min(jnp.where(cur == m, iota_e, E), axis=1, keepdims=True)
        vals.append(m)
        sels.append(idx)
        mask = mask + jnp.where(iota_e == idx, 1.0, 0.0)
        cur = jnp.where(iota_e == idx, NEGINF, cur)
    exps = [jnp.exp(v - vals[0]) for v in vals]
    denom = functools.reduce(jnp.add, exps)
    c = mask
    sh = 1
    while sh < N:
        c = c + jnp.concatenate(
            [jnp.zeros((sh, E), jnp.float32), c[: N - sh]], axis=0)
        sh *= 2
    rank = c - mask
    counts = c[N - 1:N, :]
    cc = counts
    sh = 1
    while sh < E:
        cc = cc + jnp.concatenate(
            [jnp.zeros((1, sh), jnp.float32), cc[:, : E - sh]], axis=1)
        sh *= 2
    offs = cc - counts
    posfull = offs + rank
    pos_cols = [
        jnp.sum(jnp.where(iota_e == sels[k], posfull, 0.0), axis=1,
                keepdims=True)
        for k in range(K)
    ]
    logits_ref[...] = logits
    w_ref[...] = jnp.concatenate([e / denom for e in exps], axis=1)
    pos_ref[...] = jnp.concatenate(pos_cols, axis=1).astype(jnp.int32)
    counts_ref[...] = counts.astype(jnp.int32)


def _shared_body(xb_ref, w1_ref, w2_ref, out_ref):
    i = pl.program_id(0)

    @pl.when(i == 0)
    def _():
        out_ref[...] = jnp.zeros_like(out_ref)

    xb = xb_ref[...]
    w1 = w1_ref[...].astype(jnp.bfloat16)
    h = jax.lax.dot_general(
        xb, w1, (((1,), (1,)), ((), ())), preferred_element_type=jnp.float32
    )
    h = h * jax.nn.sigmoid(h)
    w2 = w2_ref[...].astype(jnp.bfloat16)
    out_ref[...] += jax.lax.dot_general(
        h.astype(jnp.bfloat16), w2, (((1,), (1,)), ((), ())),
        preferred_element_type=jnp.float32,
    )


def _gmm_body(tile_ref, gid_ref, lo_ref, hi_ref, last_ref, xs_ref, w1_ref,
              w2_ref, out_ref, acc_ref):
    s = pl.program_id(0)
    lo = lo_ref[s]
    hi = hi_ref[s]
    tile = tile_ref[s]

    @pl.when(lo < hi)
    def _():
        xs32 = xs_ref[...]
        xlo = jax.lax.bitcast_convert_type(
            xs32 << 16, jnp.float32).astype(jnp.bfloat16)
        xhi = jax.lax.bitcast_convert_type(
            xs32 & jnp.int32(-65536), jnp.float32).astype(jnp.bfloat16)
        w1 = w1_ref[0].astype(jnp.bfloat16)
        h = jax.lax.dot_general(
            xlo, w1[:, : H // 2], (((1,), (1,)), ((), ())),
            preferred_element_type=jnp.float32)
        h += jax.lax.dot_general(
            xhi, w1[:, H // 2:], (((1,), (1,)), ((), ())),
            preferred_element_type=jnp.float32)
        h = h * jax.nn.sigmoid(h)
        o = jax.lax.dot_general(
            h.astype(jnp.bfloat16), w2_ref[0].astype(jnp.bfloat16),
            (((1,), (1,)), ((), ())), preferred_element_type=jnp.float32,
        )
        olo = jax.lax.bitcast_convert_type(
            o[:, : H // 2].astype(jnp.bfloat16).astype(jnp.float32),
            jnp.int32)
        ohi = jax.lax.bitcast_convert_type(
            o[:, H // 2:].astype(jnp.bfloat16).astype(jnp.float32),
            jnp.int32)
        o32 = ((olo >> 16) & 65535) | (ohi & jnp.int32(-65536))
        r = jax.lax.broadcasted_iota(jnp.int32, (TM, H // 2), 0) + tile * TM
        sel = (r >= lo) & (r < hi)
        acc_ref[...] = jnp.where(sel, o32, acc_ref[...])

    @pl.when(last_ref[s] == 1)
    def _():
        out_ref[...] = acc_ref[...]


def _combine_body(yg_ref, w_ref, shared_ref, out_ref):
    k = pl.program_id(1)

    @pl.when(k == 0)
    def _():
        out_ref[...] = shared_ref[...]

    iota_k = jax.lax.broadcasted_iota(jnp.int32, (w_ref.shape[0], K), 1)
    wk = jnp.sum(jnp.where(iota_k == k, w_ref[...], 0.0), axis=1,
                 keepdims=True)
    yg32 = yg_ref[...]
    flo = jax.lax.bitcast_convert_type(yg32 << 16, jnp.float32)
    fhi = jax.lax.bitcast_convert_type(yg32 & jnp.int32(-65536), jnp.float32)
    out_ref[...] += wk * jnp.concatenate([flo, fhi], axis=1)


def _sc_dispatch_rows(src, tok, pos, chunk=64):
    j = tok.shape[0]
    d = src.shape[1]
    per_w = j // NW
    mesh = plsc.VectorSubcoreMesh(core_axis_name="c", subcore_axis_name="s")

    @functools.partial(
        pl.kernel, mesh=mesh,
        out_type=jax.ShapeDtypeStruct((j, d), src.dtype),
        scratch_types=[
            pltpu.VMEM((chunk,), jnp.int32),
            pltpu.VMEM((chunk,), jnp.int32),
            pltpu.VMEM((chunk, d), src.dtype),
            pltpu.SemaphoreType.DMA,
        ],
    )
    def kern(src_hbm, tok_hbm, pos_hbm, out_hbm, tok_v, pos_v, rows_v, sem):
        wid = lax.axis_index("s") * NC + lax.axis_index("c")
        base = wid * per_w
        for c in range(per_w // chunk):
            off = base + c * chunk
            pltpu.sync_copy(tok_hbm.at[pl.ds(off, chunk)], tok_v)
            pltpu.sync_copy(pos_hbm.at[pl.ds(off, chunk)], pos_v)
            pltpu.async_copy(src_hbm.at[tok_v], rows_v, sem).wait()
            pltpu.async_copy(rows_v, out_hbm.at[pos_v], sem).wait()

    return kern(src, tok, pos)


def _sc_gather_rows(src, idx, chunk=64):
    j = idx.shape[0]
    d = src.shape[1]
    per_w = j // NW
    mesh = plsc.VectorSubcoreMesh(core_axis_name="c", subcore_axis_name="s")

    @functools.partial(
        pl.kernel, mesh=mesh,
        out_type=jax.ShapeDtypeStruct((j, d), src.dtype),
        scratch_types=[
            pltpu.VMEM((chunk,), jnp.int32),
            pltpu.VMEM((chunk, d), src.dtype),
            pltpu.SemaphoreType.DMA,
        ],
    )
    def kern(src_hbm, idx_hbm, out_hbm, idx_v, rows_v, sem):
        wid = lax.axis_index("s") * NC + lax.axis_index("c")
        base = wid * per_w
        for c in range(per_w // chunk):
            off = base + c * chunk
            pltpu.sync_copy(idx_hbm.at[pl.ds(off, chunk)], idx_v)
            pltpu.async_copy(src_hbm.at[idx_v], rows_v, sem).wait()
            pltpu.sync_copy(rows_v, out_hbm.at[pl.ds(off, chunk)])

    return kern(src, idx)


def _metadata(counts):
    ends = jnp.cumsum(counts)
    starts = ends - counts
    first_tile = starts // TM
    last_tile = jnp.where(counts > 0, (ends - 1) // TM, 0)
    span = jnp.where(counts > 0, last_tile - first_tile + 1, 0)
    cum = jnp.cumsum(span)
    total = cum[-1]
    sidx = jnp.arange(NSTEPS, dtype=jnp.int32)
    gid = jnp.searchsorted(cum, sidx, side="right").astype(jnp.int32)
    valid = sidx < total
    gid_last = jnp.searchsorted(cum, total - 1, side="right").astype(jnp.int32)
    gidc = jnp.where(valid, jnp.minimum(gid, E - 1), gid_last)
    local = sidx - (cum[gidc] - span[gidc])
    tile = jnp.where(valid, first_tile[gidc] + local, NTILES - 1).astype(jnp.int32)
    lo = jnp.where(valid, jnp.maximum(starts[gidc], tile * TM), 0).astype(jnp.int32)
    hi = jnp.where(valid, jnp.minimum(ends[gidc], (tile + 1) * TM), 0).astype(jnp.int32)
    tile_next = jnp.concatenate([tile[1:], tile[:1]])
    last = jnp.where(
        valid, ((sidx + 1 == total) | (tile_next != tile)).astype(jnp.int32), 0
    ).astype(jnp.int32)
    return tile, gidc, lo, hi, last


def kernel(hidden_states, Wg, W1s, W2s, W1, W2):
    b, s, h = hidden_states.shape
    x = hidden_states.reshape(-1, h)
    xb = x.astype(jnp.bfloat16)

    logits, w, pos, counts = pl.pallas_call(
        _router_body,
        out_shape=(
            jax.ShapeDtypeStruct((N, E), jnp.float32),
            jax.ShapeDtypeStruct((N, K), jnp.float32),
            jax.ShapeDtypeStruct((N, K), jnp.int32),
            jax.ShapeDtypeStruct((1, E), jnp.int32),
        ),
    )(x, Wg)

    IC = 512
    shared = pl.pallas_call(
        _shared_body,
        grid=(I // IC,),
        in_specs=[
            pl.BlockSpec((N, H), lambda i: (0, 0)),
            pl.BlockSpec((IC, H), lambda i: (i, 0)),
            pl.BlockSpec((H, IC), lambda i: (0, i)),
        ],
        out_specs=pl.BlockSpec((N, H), lambda i: (0, 0)),
        out_shape=jax.ShapeDtypeStruct((N, H), jnp.float32),
    )(xb, W1s, W2s)

    tok = jnp.arange(NK, dtype=jnp.int32) // K
    pos_flat = pos.reshape(NK)
    lo16 = jax.lax.bitcast_convert_type(
        xb[:, : H // 2], jnp.uint16).astype(jnp.uint32)
    hi16 = jax.lax.bitcast_convert_type(
        xb[:, H // 2:], jnp.uint16).astype(jnp.uint32)
    xb32 = jax.lax.bitcast_convert_type(lo16 | (hi16 << 16), jnp.int32)
    xs = _sc_dispatch_rows(xb32, tok, pos_flat)

    tile, gid, lo, hi, lastf = _metadata(counts[0])
    ys = pl.pallas_call(
        _gmm_body,
        grid_spec=pltpu.PrefetchScalarGridSpec(
            num_scalar_prefetch=5,
            grid=(NSTEPS,),
            in_specs=[
                pl.BlockSpec((TM, H // 2),
                             lambda st, t, g, l, hh, lf: (t[st], 0)),
                pl.BlockSpec((1, F, H),
                             lambda st, t, g, l, hh, lf: (g[st], 0, 0)),
                pl.BlockSpec((1, H, F),
                             lambda st, t, g, l, hh, lf: (g[st], 0, 0)),
            ],
            out_specs=pl.BlockSpec((TM, H // 2),
                                   lambda st, t, g, l, hh, lf: (t[st], 0)),
            scratch_shapes=[pltpu.VMEM((TM, H // 2), jnp.int32)],
        ),
        out_shape=jax.ShapeDtypeStruct((NK, H // 2), jnp.int32),
    )(tile, gid, lo, hi, lastf, xs, W1, W2)

    pos_kmaj = pos.T.reshape(NK)
    yg = _sc_gather_rows(ys, pos_kmaj)

    TN = 256
    out = pl.pallas_call(
        _combine_body,
        grid=(N // TN, K),
        in_specs=[
            pl.BlockSpec((TN, H // 2), lambda t, k: (k * (N // TN) + t, 0)),
            pl.BlockSpec((TN, K), lambda t, k: (t, 0)),
            pl.BlockSpec((TN, H), lambda t, k: (t, 0)),
        ],
        out_specs=pl.BlockSpec((TN, H), lambda t, k: (t, 0)),
        out_shape=jax.ShapeDtypeStruct((N, H), jnp.float32),
    )(yg, w, shared)

    return (out.reshape(b, s, h), logits.reshape(b, s, E))
